# baseline (speedup 1.0000x reference)
import functools

import jax
import jax.numpy as jnp
from jax import lax
from jax.experimental import pallas as pl
from jax.experimental.pallas import tpu as pltpu
from jax.experimental.pallas import tpu_sc as plsc

N = 10000
E = 320000
D_IN = 128
D_H = 64
G = 64

NC = 2
NS = 16
NT = NC * NS
EPT = E // NT
CH = 80
U = 7
EP = 322560
EPTP = EP // NT
NCHUNK = EPTP // CH
NGRP = NCHUNK // U
NPAIR = NGRP // 2
NPAD = 10240
RPT = NPAD // NS
ZR = 128

_mesh = plsc.VectorSubcoreMesh(core_axis_name="c", subcore_axis_name="s")
_sc_params = pltpu.CompilerParams(use_tc_tiling_on_sc=False)


def _zero_fill(ref, nrows, width):
    def body(r, carry):
        for cix in range(width // 16):
            ref[r, pl.ds(cix * 16, 16)] = jnp.zeros((16,), jnp.float32)
        return carry
    lax.fori_loop(0, nrows, body, 0)


@functools.partial(
    pl.kernel,
    out_type=jax.ShapeDtypeStruct((NC, NPAD, 16), jnp.float32),
    mesh=_mesh,
    scratch_types=[
        [pltpu.VMEM((CH,), jnp.int32)] * (2 * U),
        pltpu.VMEM((CH, 16), jnp.float32),
        pltpu.VMEM((ZR, 16), jnp.float32),
        pltpu.VMEM_SHARED((NPAD, 16), jnp.float32),
        pltpu.SemaphoreType.DMA,
        pltpu.SemaphoreType.DMA,
    ],
    compiler_params=_sc_params,
)
def _sc_deg(dst_hbm, out_hbm, didx, ones_v, zbuf, dacc, isem, ssem):
    c = lax.axis_index("c")
    s = lax.axis_index("s")
    t = c * NS + s

    def fill_ones(r, carry):
        ones_v[r, :] = jnp.ones((16,), jnp.float32)
        return carry
    lax.fori_loop(0, CH, fill_ones, 0)
    _zero_fill(zbuf, ZR, 16)
    for j in range(RPT // ZR):
        pltpu.sync_copy(zbuf, dacc.at[pl.ds(s * RPT + j * ZR, ZR)])
    plsc.subcore_barrier()

    ebase = t * EPTP

    def fire_idx(bank, gidx):
        b = ebase + gidx * U * CH
        return [pltpu.async_copy(dst_hbm.at[pl.ds(b + k * CH, CH)],
                                 didx[bank * U + k], isem)
                for k in range(U)]

    def fire_scat(bank):
        return [pltpu.async_copy(ones_v, dacc.at[didx[bank * U + k]],
                                 ssem, add=True)
                for k in range(U)]

    def drain(ds):
        for d in ds:
            d.wait()

    def pair(m, carry):
        ia = fire_idx(0, 2 * m)
        drain(ia)
        sa = fire_scat(0)
        ib = fire_idx(1, 2 * m + 1)
        drain(ib)
        sb = fire_scat(1)
        drain(sa)
        drain(sb)
        return carry
    lax.fori_loop(0, NPAIR, pair, 0)
    plsc.subcore_barrier()
    pltpu.sync_copy(dacc.at[pl.ds(s * RPT, RPT)],
                    out_hbm.at[c, pl.ds(s * RPT, RPT)])


@functools.partial(
    pl.kernel,
    out_type=jax.ShapeDtypeStruct((NC, NPAD, D_H), jnp.float32),
    mesh=_mesh,
    scratch_types=[
        [pltpu.VMEM((CH,), jnp.int32)] * (2 * U),
        [pltpu.VMEM((CH,), jnp.int32)] * (2 * U),
        [pltpu.VMEM((CH, D_H), jnp.float32)] * (2 * U),
        pltpu.VMEM((ZR, D_H), jnp.float32),
        pltpu.VMEM_SHARED((NPAD, D_H), jnp.float32),
        pltpu.SemaphoreType.DMA,
        pltpu.SemaphoreType.DMA,
        pltpu.SemaphoreType.DMA,
    ],
    compiler_params=_sc_params,
)
def _sc_agg(hs_hbm, src_hbm, dst_hbm, out_hbm, sidx, didx, rows, zbuf, acc,
            isem, gsem, ssem):
    c = lax.axis_index("c")
    s = lax.axis_index("s")
    t = c * NS + s

    _zero_fill(zbuf, ZR, D_H)
    for j in range(RPT // ZR):
        pltpu.sync_copy(zbuf, acc.at[pl.ds(s * RPT + j * ZR, ZR)])
    plsc.subcore_barrier()

    ebase = t * EPTP

    def fire_idx(bank, gidx):
        b = ebase + gidx * U * CH
        out = []
        for k in range(U):
            out.append(pltpu.async_copy(src_hbm.at[pl.ds(b + k * CH, CH)],
                                        sidx[bank * U + k], isem))
            out.append(pltpu.async_copy(dst_hbm.at[pl.ds(b + k * CH, CH)],
                                        didx[bank * U + k], isem))
        return out

    def fire_gath(bank):
        return [pltpu.async_copy(hs_hbm.at[sidx[bank * U + k]],
                                 rows[bank * U + k], gsem)
                for k in range(U)]

    def fire_scat(bank):
        return [pltpu.async_copy(rows[bank * U + k],
                                 acc.at[didx[bank * U + k]], ssem,
                                 add=True)
                for k in range(U)]

    def drain(ds):
        for d in ds:
            d.wait()

    def pair(m, carry):
        ia = fire_idx(0, 2 * m)
        drain(ia)
        ga = fire_gath(0)
        ib = fire_idx(1, 2 * m + 1)
        drain(ga)
        sa = fire_scat(0)
        drain(sa)
        drain(ib)
        gb = fire_gath(1)
        drain(gb)
        sb = fire_scat(1)
        drain(sb)
        return carry
    lax.fori_loop(0, NPAIR, pair, 0)
    plsc.subcore_barrier()
    pltpu.sync_copy(acc.at[pl.ds(s * RPT, RPT)],
                    out_hbm.at[c, pl.ds(s * RPT, RPT)])


def _dis_from_degp(degp_ref):
    deg = 1.0 + degp_ref[0, 0:N, 0:1] + degp_ref[1, 0:N, 0:1]
    return lax.rsqrt(deg)


def _tc_first(x_ref, w1_ref, degp_ref, hs_ref):
    dis = _dis_from_degp(degp_ref)
    h = lax.dot_general(x_ref[...], w1_ref[...], (((1,), (0,)), ((), ())),
                        preferred_element_type=jnp.float32,
                        precision=lax.Precision.HIGHEST)
    hs_ref[...] = h * dis


def _tc_mid(hs_ref, p_ref, degp_ref, b1_ref, w2_ref, hs2_ref):
    dis = _dis_from_degp(degp_ref)
    ssum = (hs_ref[...] + p_ref[0, 0:N, :] + p_ref[1, 0:N, :])
    r1 = jnp.maximum(dis * ssum + b1_ref[...], 0.0)
    h2 = lax.dot_general(r1, w2_ref[...], (((1,), (0,)), ((), ())),
                         preferred_element_type=jnp.float32,
                         precision=lax.Precision.HIGHEST)
    hs2_ref[...] = h2 * dis


def _tc_last(hs2_ref, p_ref, degp_ref, b2_ref, batch_ref, wfc_ref, bfc_ref,
             out_ref):
    dis = _dis_from_degp(degp_ref)
    ssum = (hs2_ref[...] + p_ref[0, 0:N, :] + p_ref[1, 0:N, :])
    r2 = jnp.maximum(dis * ssum + b2_ref[...], 0.0)
    gid = lax.broadcasted_iota(jnp.int32, (N, G), 1)
    oneh = (batch_ref[...] == gid).astype(jnp.float32)
    sums = lax.dot_general(oneh, r2, (((0,), (0,)), ((), ())),
                           preferred_element_type=jnp.float32,
                           precision=lax.Precision.HIGHEST)
    ones_col = jnp.ones((N, 1), jnp.float32)
    cnts = lax.dot_general(oneh, ones_col, (((0,), (0,)), ((), ())),
                           preferred_element_type=jnp.float32,
                           precision=lax.Precision.HIGHEST)
    g = sums / jnp.maximum(cnts, 1.0)
    out_ref[...] = lax.dot_general(g, wfc_ref[...], (((1,), (0,)), ((), ())),
                                   preferred_element_type=jnp.float32,
                                   precision=lax.Precision.HIGHEST) + bfc_ref[...]


_first = pl.pallas_call(
    _tc_first, out_shape=jax.ShapeDtypeStruct((N, D_H), jnp.float32))
_mid = pl.pallas_call(
    _tc_mid, out_shape=jax.ShapeDtypeStruct((N, D_H), jnp.float32))
_last = pl.pallas_call(
    _tc_last, out_shape=jax.ShapeDtypeStruct((G, 1), jnp.float32))


def kernel(x, edge_index, batch, W1, b1, W2, b2, Wfc, bfc):
    ei = edge_index.astype(jnp.int32)
    src = jnp.concatenate([ei[0], jnp.zeros((EP - E,), jnp.int32)])
    dst = jnp.concatenate([ei[1], jnp.full((EP - E,), NPAD - 1, jnp.int32)])
    degp = _sc_deg(dst)
    hs1 = _first(x, W1, degp)
    p1 = _sc_agg(hs1, src, dst)
    hs2 = _mid(hs1, p1, degp, b1.reshape(1, D_H), W2)
    p2 = _sc_agg(hs2, src, dst)
    out = _last(hs2, p2, degp, b2.reshape(1, D_H),
                batch.astype(jnp.int32).reshape(N, 1), Wfc,
                bfc.reshape(1, 1))
    return out.reshape(G)


# speedup vs baseline: 1.2703x; 1.2703x over previous
import functools

import jax
import jax.numpy as jnp
from jax import lax
from jax.experimental import pallas as pl
from jax.experimental.pallas import tpu as pltpu
from jax.experimental.pallas import tpu_sc as plsc

N = 10000
E = 320000
D_IN = 128
D_H = 64
G = 64

NC = 2
NS = 16
NT = NC * NS
EPT = E // NT
CH = 80
U = 5
NCHUNK = EPT // CH
NGRP = NCHUNK // U
NPAD = 10240
RPT = NPAD // NS
ZR = 128

_mesh = plsc.VectorSubcoreMesh(core_axis_name="c", subcore_axis_name="s")
_sc_params = pltpu.CompilerParams(use_tc_tiling_on_sc=False)


def _zero_fill(ref, nrows, width):
    def body(r, carry):
        for cix in range(width // 16):
            ref[r, pl.ds(cix * 16, 16)] = jnp.zeros((16,), jnp.float32)
        return carry
    lax.fori_loop(0, nrows, body, 0)


@functools.partial(
    pl.kernel,
    out_type=jax.ShapeDtypeStruct((NC, NPAD, 16), jnp.float32),
    mesh=_mesh,
    scratch_types=[
        [pltpu.VMEM((CH,), jnp.int32)] * U,
        pltpu.VMEM((CH, 16), jnp.float32),
        pltpu.VMEM((ZR, 16), jnp.float32),
        pltpu.VMEM_SHARED((NPAD, 16), jnp.float32),
        pltpu.SemaphoreType.DMA,
        pltpu.SemaphoreType.DMA,
    ],
    compiler_params=_sc_params,
)
def _sc_deg(dst_hbm, out_hbm, didx, ones_v, zbuf, dacc, isem, ssem):
    c = lax.axis_index("c")
    s = lax.axis_index("s")
    t = c * NS + s

    def fill_ones(r, carry):
        ones_v[r, :] = jnp.ones((16,), jnp.float32)
        return carry
    lax.fori_loop(0, CH, fill_ones, 0)
    _zero_fill(zbuf, ZR, 16)
    for j in range(RPT // ZR):
        pltpu.sync_copy(zbuf, dacc.at[pl.ds(s * RPT + j * ZR, ZR)])
    plsc.subcore_barrier()

    ebase = t * EPT

    def group(g, carry):
        b = ebase + g * U * CH
        ia = [pltpu.async_copy(dst_hbm.at[pl.ds(b + k * CH, CH)],
                               didx[k], isem) for k in range(U)]
        for d in ia:
            d.wait()
        sa = [pltpu.async_copy(ones_v, dacc.at[didx[k]], ssem, add=True)
              for k in range(U)]
        for d in sa:
            d.wait()
        return carry
    lax.fori_loop(0, NGRP, group, 0)
    plsc.subcore_barrier()
    pltpu.sync_copy(dacc.at[pl.ds(s * RPT, RPT)],
                    out_hbm.at[c, pl.ds(s * RPT, RPT)])


@functools.partial(
    pl.kernel,
    out_type=jax.ShapeDtypeStruct((NC, NPAD, D_H), jnp.float32),
    mesh=_mesh,
    scratch_types=[
        [pltpu.VMEM((CH,), jnp.int32)] * U,
        [pltpu.VMEM((CH,), jnp.int32)] * U,
        [pltpu.VMEM((CH, D_H), jnp.float32)] * U,
        pltpu.VMEM((ZR, D_H), jnp.float32),
        pltpu.VMEM_SHARED((NPAD, D_H), jnp.float32),
        pltpu.SemaphoreType.DMA,
        pltpu.SemaphoreType.DMA,
        pltpu.SemaphoreType.DMA,
    ],
    compiler_params=_sc_params,
)
def _sc_agg(hs_hbm, src_hbm, dst_hbm, out_hbm, sidx, didx, rows, zbuf, acc,
            isem, gsem, ssem):
    c = lax.axis_index("c")
    s = lax.axis_index("s")
    t = c * NS + s

    _zero_fill(zbuf, ZR, D_H)
    for j in range(RPT // ZR):
        pltpu.sync_copy(zbuf, acc.at[pl.ds(s * RPT + j * ZR, ZR)])
    plsc.subcore_barrier()

    ebase = t * EPT

    def group(g, carry):
        b = ebase + g * U * CH
        ia = []
        for k in range(U):
            ia.append(pltpu.async_copy(src_hbm.at[pl.ds(b + k * CH, CH)],
                                       sidx[k], isem))
            ia.append(pltpu.async_copy(dst_hbm.at[pl.ds(b + k * CH, CH)],
                                       didx[k], isem))
        for d in ia:
            d.wait()
        ga = [pltpu.async_copy(hs_hbm.at[sidx[k]], rows[k], gsem)
              for k in range(U)]
        for d in ga:
            d.wait()
        sa = [pltpu.async_copy(rows[k], acc.at[didx[k]], ssem, add=True)
              for k in range(U)]
        for d in sa:
            d.wait()
        return carry
    lax.fori_loop(0, NGRP, group, 0)
    plsc.subcore_barrier()
    pltpu.sync_copy(acc.at[pl.ds(s * RPT, RPT)],
                    out_hbm.at[c, pl.ds(s * RPT, RPT)])


def _dis_from_degp(degp_ref):
    deg = 1.0 + degp_ref[0, 0:N, 0:1] + degp_ref[1, 0:N, 0:1]
    return lax.rsqrt(deg)


def _tc_matmul1(x_ref, w1_ref, h_ref):
    h_ref[...] = lax.dot_general(x_ref[...], w1_ref[...],
                                 (((1,), (0,)), ((), ())),
                                 preferred_element_type=jnp.float32,
                                 precision=lax.Precision.HIGHEST)


def _tc_scale1(h_ref, degp_ref, hs_ref):
    hs_ref[...] = h_ref[...] * _dis_from_degp(degp_ref)


def _tc_mid(hs_ref, p_ref, degp_ref, b1_ref, w2_ref, hs2_ref):
    dis = _dis_from_degp(degp_ref)
    ssum = (hs_ref[...] + p_ref[0, 0:N, :] + p_ref[1, 0:N, :])
    r1 = jnp.maximum(dis * ssum + b1_ref[...], 0.0)
    h2 = lax.dot_general(r1, w2_ref[...], (((1,), (0,)), ((), ())),
                         preferred_element_type=jnp.float32,
                         precision=lax.Precision.HIGHEST)
    hs2_ref[...] = h2 * dis


def _tc_last(hs2_ref, p_ref, degp_ref, b2_ref, batch_ref, wfc_ref, bfc_ref,
             out_ref):
    dis = _dis_from_degp(degp_ref)
    ssum = (hs2_ref[...] + p_ref[0, 0:N, :] + p_ref[1, 0:N, :])
    r2 = jnp.maximum(dis * ssum + b2_ref[...], 0.0)
    gid = lax.broadcasted_iota(jnp.int32, (N, G), 1)
    oneh = (batch_ref[...] == gid).astype(jnp.float32)
    sums = lax.dot_general(oneh, r2, (((0,), (0,)), ((), ())),
                           preferred_element_type=jnp.float32,
                           precision=lax.Precision.HIGHEST)
    ones_col = jnp.ones((N, 1), jnp.float32)
    cnts = lax.dot_general(oneh, ones_col, (((0,), (0,)), ((), ())),
                           preferred_element_type=jnp.float32,
                           precision=lax.Precision.HIGHEST)
    g = sums / jnp.maximum(cnts, 1.0)
    out_ref[...] = lax.dot_general(g, wfc_ref[...], (((1,), (0,)), ((), ())),
                                   preferred_element_type=jnp.float32,
                                   precision=lax.Precision.HIGHEST) + bfc_ref[...]


_matmul1 = pl.pallas_call(
    _tc_matmul1, out_shape=jax.ShapeDtypeStruct((N, D_H), jnp.float32))
_scale1 = pl.pallas_call(
    _tc_scale1, out_shape=jax.ShapeDtypeStruct((N, D_H), jnp.float32))
_mid = pl.pallas_call(
    _tc_mid, out_shape=jax.ShapeDtypeStruct((N, D_H), jnp.float32))
_last = pl.pallas_call(
    _tc_last, out_shape=jax.ShapeDtypeStruct((G, 1), jnp.float32))


def kernel(x, edge_index, batch, W1, b1, W2, b2, Wfc, bfc):
    ei = edge_index.astype(jnp.int32)
    src = ei[0]
    dst = ei[1]
    degp = _sc_deg(dst)
    h1 = _matmul1(x, W1)
    hs1 = _scale1(h1, degp)
    p1 = _sc_agg(hs1, src, dst)
    hs2 = _mid(hs1, p1, degp, b1.reshape(1, D_H), W2)
    p2 = _sc_agg(hs2, src, dst)
    out = _last(hs2, p2, degp, b2.reshape(1, D_H),
                batch.astype(jnp.int32).reshape(N, 1), Wfc,
                bfc.reshape(1, 1))
    return out.reshape(G)


# speedup vs baseline: 1.3152x; 1.0354x over previous
import functools

import jax
import jax.numpy as jnp
from jax import lax
from jax.experimental import pallas as pl
from jax.experimental.pallas import tpu as pltpu
from jax.experimental.pallas import tpu_sc as plsc

N = 10000
E = 320000
D_IN = 128
D_H = 64
G = 64

NC = 2
NS = 16
NT = NC * NS
EPT = E // NT
CH = 80
U = 5
NCHUNK = EPT // CH
NGRP = NCHUNK // U
NPAD = 10240
RPT = NPAD // NS
ZR = 128

_mesh = plsc.VectorSubcoreMesh(core_axis_name="c", subcore_axis_name="s")
_sc_params = pltpu.CompilerParams(use_tc_tiling_on_sc=False)


def _zero_fill(ref, nrows, width):
    def body(r, carry):
        for cix in range(width // 16):
            ref[r, pl.ds(cix * 16, 16)] = jnp.zeros((16,), jnp.float32)
        return carry
    lax.fori_loop(0, nrows, body, 0)


@functools.partial(
    pl.kernel,
    out_type=jax.ShapeDtypeStruct((NC, NPAD, 16), jnp.float32),
    mesh=_mesh,
    scratch_types=[
        [pltpu.VMEM((CH,), jnp.int32)] * U,
        pltpu.VMEM((CH, 16), jnp.float32),
        pltpu.VMEM((ZR, 16), jnp.float32),
        pltpu.VMEM_SHARED((NPAD, 16), jnp.float32),
        pltpu.SemaphoreType.DMA,
        pltpu.SemaphoreType.DMA,
    ],
    compiler_params=_sc_params,
)
def _sc_deg(ei_hbm, out_hbm, didx, ones_v, zbuf, dacc, isem, ssem):
    c = lax.axis_index("c")
    s = lax.axis_index("s")
    t = c * NS + s

    def fill_ones(r, carry):
        ones_v[r, :] = jnp.ones((16,), jnp.float32)
        return carry
    lax.fori_loop(0, CH, fill_ones, 0)
    _zero_fill(zbuf, ZR, 16)
    for j in range(RPT // ZR):
        pltpu.sync_copy(zbuf, dacc.at[pl.ds(s * RPT + j * ZR, ZR)])
    plsc.subcore_barrier()

    ebase = t * EPT

    def group(g, carry):
        b = ebase + g * U * CH
        ia = [pltpu.async_copy(ei_hbm.at[1, pl.ds(b + k * CH, CH)],
                               didx[k], isem) for k in range(U)]
        for d in ia:
            d.wait()
        sa = [pltpu.async_copy(ones_v, dacc.at[didx[k]], ssem, add=True)
              for k in range(U)]
        for d in sa:
            d.wait()
        return carry
    lax.fori_loop(0, NGRP, group, 0)
    plsc.subcore_barrier()
    pltpu.sync_copy(dacc.at[pl.ds(s * RPT, RPT)],
                    out_hbm.at[c, pl.ds(s * RPT, RPT)])


@functools.partial(
    pl.kernel,
    out_type=jax.ShapeDtypeStruct((NC, NPAD, D_H), jnp.float32),
    mesh=_mesh,
    scratch_types=[
        [pltpu.VMEM((CH,), jnp.int32)] * U,
        [pltpu.VMEM((CH,), jnp.int32)] * U,
        [pltpu.VMEM((CH, D_H), jnp.float32)] * U,
        pltpu.VMEM((ZR, D_H), jnp.float32),
        pltpu.VMEM_SHARED((NPAD, D_H), jnp.float32),
        pltpu.SemaphoreType.DMA,
        pltpu.SemaphoreType.DMA,
        pltpu.SemaphoreType.DMA,
    ],
    compiler_params=_sc_params,
)
def _sc_agg(hs_hbm, ei_hbm, out_hbm, sidx, didx, rows, zbuf, acc,
            isem, gsem, ssem):
    c = lax.axis_index("c")
    s = lax.axis_index("s")
    t = c * NS + s

    _zero_fill(zbuf, ZR, D_H)
    for j in range(RPT // ZR):
        pltpu.sync_copy(zbuf, acc.at[pl.ds(s * RPT + j * ZR, ZR)])
    plsc.subcore_barrier()

    ebase = t * EPT

    def group(g, carry):
        b = ebase + g * U * CH
        ia = []
        for k in range(U):
            ia.append(pltpu.async_copy(ei_hbm.at[0, pl.ds(b + k * CH, CH)],
                                       sidx[k], isem))
            ia.append(pltpu.async_copy(ei_hbm.at[1, pl.ds(b + k * CH, CH)],
                                       didx[k], isem))
        for d in ia:
            d.wait()
        ga = [pltpu.async_copy(hs_hbm.at[sidx[k]], rows[k], gsem)
              for k in range(U)]
        for d in ga:
            d.wait()
        sa = [pltpu.async_copy(rows[k], acc.at[didx[k]], ssem, add=True)
              for k in range(U)]
        for d in sa:
            d.wait()
        return carry
    lax.fori_loop(0, NGRP, group, 0)
    plsc.subcore_barrier()
    pltpu.sync_copy(acc.at[pl.ds(s * RPT, RPT)],
                    out_hbm.at[c, pl.ds(s * RPT, RPT)])


def _dis_from_degp(degp_ref):
    deg = 1.0 + degp_ref[0, 0:N, 0:1] + degp_ref[1, 0:N, 0:1]
    return lax.rsqrt(deg)


def _tc_matmul1(x_ref, w1_ref, h_ref):
    h_ref[...] = lax.dot_general(x_ref[...], w1_ref[...],
                                 (((1,), (0,)), ((), ())),
                                 preferred_element_type=jnp.float32,
                                 precision=lax.Precision.HIGHEST)


def _tc_scale1(h_ref, degp_ref, hs_ref):
    hs_ref[...] = h_ref[...] * _dis_from_degp(degp_ref)


def _tc_mid(hs_ref, p_ref, degp_ref, b1_ref, w2_ref, hs2_ref):
    dis = _dis_from_degp(degp_ref)
    ssum = (hs_ref[...] + p_ref[0, 0:N, :] + p_ref[1, 0:N, :])
    r1 = jnp.maximum(dis * ssum + b1_ref[...], 0.0)
    h2 = lax.dot_general(r1, w2_ref[...], (((1,), (0,)), ((), ())),
                         preferred_element_type=jnp.float32,
                         precision=lax.Precision.HIGHEST)
    hs2_ref[...] = h2 * dis


def _tc_last(hs2_ref, p_ref, degp_ref, b2_ref, batch_ref, wfc_ref, bfc_ref,
             out_ref):
    dis = _dis_from_degp(degp_ref)
    ssum = (hs2_ref[...] + p_ref[0, 0:N, :] + p_ref[1, 0:N, :])
    r2 = jnp.maximum(dis * ssum + b2_ref[...], 0.0)
    gid = lax.broadcasted_iota(jnp.int32, (N, G), 1)
    oneh = (batch_ref[...] == gid).astype(jnp.float32)
    sums = lax.dot_general(oneh, r2, (((0,), (0,)), ((), ())),
                           preferred_element_type=jnp.float32,
                           precision=lax.Precision.HIGHEST)
    ones_col = jnp.ones((N, 1), jnp.float32)
    cnts = lax.dot_general(oneh, ones_col, (((0,), (0,)), ((), ())),
                           preferred_element_type=jnp.float32,
                           precision=lax.Precision.HIGHEST)
    g = sums / jnp.maximum(cnts, 1.0)
    out_ref[...] = lax.dot_general(g, wfc_ref[...], (((1,), (0,)), ((), ())),
                                   preferred_element_type=jnp.float32,
                                   precision=lax.Precision.HIGHEST) + bfc_ref[...]


_matmul1 = pl.pallas_call(
    _tc_matmul1, out_shape=jax.ShapeDtypeStruct((N, D_H), jnp.float32))
_scale1 = pl.pallas_call(
    _tc_scale1, out_shape=jax.ShapeDtypeStruct((N, D_H), jnp.float32))
_mid = pl.pallas_call(
    _tc_mid, out_shape=jax.ShapeDtypeStruct((N, D_H), jnp.float32))
_last = pl.pallas_call(
    _tc_last, out_shape=jax.ShapeDtypeStruct((G, 1), jnp.float32))


def kernel(x, edge_index, batch, W1, b1, W2, b2, Wfc, bfc):
    ei = edge_index.astype(jnp.int32)
    degp = _sc_deg(ei)
    h1 = _matmul1(x, W1)
    hs1 = _scale1(h1, degp)
    p1 = _sc_agg(hs1, ei)
    hs2 = _mid(hs1, p1, degp, b1.reshape(1, D_H), W2)
    p2 = _sc_agg(hs2, ei)
    out = _last(hs2, p2, degp, b2.reshape(1, D_H),
                batch.astype(jnp.int32).reshape(N, 1), Wfc,
                bfc.reshape(1, 1))
    return out.reshape(G)


# speedup vs baseline: 1.4063x; 1.0693x over previous
import functools

import jax
import jax.numpy as jnp
from jax import lax
from jax.experimental import pallas as pl
from jax.experimental.pallas import tpu as pltpu
from jax.experimental.pallas import tpu_sc as plsc

N = 10000
E = 320000
D_IN = 128
D_H = 64
G = 64

NC = 2
NS = 16
NT = NC * NS
EPT = E // NT
CH = 80
U = 5
NCHUNK = EPT // CH
NGRP = NCHUNK // U
NPAD = 10240
RPT = NPAD // NS
ZR = 128

_mesh = plsc.VectorSubcoreMesh(core_axis_name="c", subcore_axis_name="s")
_sc_params = pltpu.CompilerParams(use_tc_tiling_on_sc=False)


def _zero_fill(ref, nrows, width):
    def body(r, carry):
        for cix in range(width // 16):
            ref[r, pl.ds(cix * 16, 16)] = jnp.zeros((16,), jnp.float32)
        return carry
    lax.fori_loop(0, nrows, body, 0)


@functools.partial(
    pl.kernel,
    out_type=jax.ShapeDtypeStruct((NC, NPAD), jnp.float32),
    mesh=_mesh,
    scratch_types=[
        [pltpu.VMEM((CH,), jnp.int32)] * U,
        pltpu.VMEM((CH,), jnp.float32),
        pltpu.VMEM((RPT,), jnp.float32),
        pltpu.VMEM_SHARED((NPAD,), jnp.float32),
        pltpu.SemaphoreType.DMA,
        pltpu.SemaphoreType.DMA,
    ],
    compiler_params=_sc_params,
)
def _sc_deg(ei_hbm, out_hbm, didx, ones_v, zbuf, dacc, isem, ssem):
    c = lax.axis_index("c")
    s = lax.axis_index("s")
    t = c * NS + s

    def fill_ones(r, carry):
        ones_v[pl.ds(r * 16, 16)] = jnp.ones((16,), jnp.float32)
        return carry
    lax.fori_loop(0, CH // 16, fill_ones, 0)

    def fill_zero(r, carry):
        zbuf[pl.ds(r * 16, 16)] = jnp.zeros((16,), jnp.float32)
        return carry
    lax.fori_loop(0, RPT // 16, fill_zero, 0)
    pltpu.sync_copy(zbuf, dacc.at[pl.ds(s * RPT, RPT)])
    plsc.subcore_barrier()

    ebase = t * EPT

    def group(g, carry):
        b = ebase + g * U * CH
        ia = [pltpu.async_copy(ei_hbm.at[1, pl.ds(b + k * CH, CH)],
                               didx[k], isem) for k in range(U)]
        for d in ia:
            d.wait()
        sa = [pltpu.async_copy(ones_v, dacc.at[didx[k]], ssem, add=True)
              for k in range(U)]
        for d in sa:
            d.wait()
        return carry
    lax.fori_loop(0, NGRP, group, 0)
    plsc.subcore_barrier()
    pltpu.sync_copy(dacc.at[pl.ds(s * RPT, RPT)],
                    out_hbm.at[c, pl.ds(s * RPT, RPT)])


@functools.partial(
    pl.kernel,
    out_type=jax.ShapeDtypeStruct((NC, NPAD, D_H), jnp.float32),
    mesh=_mesh,
    scratch_types=[
        [pltpu.VMEM((CH,), jnp.int32)] * U,
        [pltpu.VMEM((CH,), jnp.int32)] * U,
        [pltpu.VMEM((CH, D_H), jnp.float32)] * U,
        pltpu.VMEM((ZR, D_H), jnp.float32),
        pltpu.VMEM_SHARED((NPAD, D_H), jnp.float32),
        pltpu.SemaphoreType.DMA,
        pltpu.SemaphoreType.DMA,
        pltpu.SemaphoreType.DMA,
    ],
    compiler_params=_sc_params,
)
def _sc_agg(hs_hbm, ei_hbm, out_hbm, sidx, didx, rows, zbuf, acc,
            isem, gsem, ssem):
    c = lax.axis_index("c")
    s = lax.axis_index("s")
    t = c * NS + s

    _zero_fill(zbuf, ZR, D_H)
    for j in range(RPT // ZR):
        pltpu.sync_copy(zbuf, acc.at[pl.ds(s * RPT + j * ZR, ZR)])
    plsc.subcore_barrier()

    ebase = t * EPT

    def group(g, carry):
        b = ebase + g * U * CH
        ia = []
        for k in range(U):
            ia.append(pltpu.async_copy(ei_hbm.at[0, pl.ds(b + k * CH, CH)],
                                       sidx[k], isem))
            ia.append(pltpu.async_copy(ei_hbm.at[1, pl.ds(b + k * CH, CH)],
                                       didx[k], isem))
        for d in ia:
            d.wait()
        ga = [pltpu.async_copy(hs_hbm.at[sidx[k]], rows[k], gsem)
              for k in range(U)]
        for d in ga:
            d.wait()
        sa = [pltpu.async_copy(rows[k], acc.at[didx[k]], ssem, add=True)
              for k in range(U)]
        for d in sa:
            d.wait()
        return carry
    lax.fori_loop(0, NGRP, group, 0)
    plsc.subcore_barrier()
    pltpu.sync_copy(acc.at[pl.ds(s * RPT, RPT)],
                    out_hbm.at[c, pl.ds(s * RPT, RPT)])


def _dis_from_degp(degp_ref):
    deg = 1.0 + degp_ref[0, 0:N] + degp_ref[1, 0:N]
    return lax.rsqrt(deg).reshape(N, 1)


def _tc_matmul1(x_ref, w1_ref, h_ref):
    h_ref[...] = lax.dot_general(x_ref[...], w1_ref[...],
                                 (((1,), (0,)), ((), ())),
                                 preferred_element_type=jnp.float32,
                                 precision=lax.Precision.HIGHEST)


def _tc_scale1(h_ref, degp_ref, hs_ref):
    hs_ref[...] = h_ref[...] * _dis_from_degp(degp_ref)


def _tc_mid(hs_ref, p_ref, degp_ref, b1_ref, w2_ref, hs2_ref):
    dis = _dis_from_degp(degp_ref)
    ssum = (hs_ref[...] + p_ref[0, 0:N, :] + p_ref[1, 0:N, :])
    r1 = jnp.maximum(dis * ssum + b1_ref[...], 0.0)
    h2 = lax.dot_general(r1, w2_ref[...], (((1,), (0,)), ((), ())),
                         preferred_element_type=jnp.float32,
                         precision=lax.Precision.HIGHEST)
    hs2_ref[...] = h2 * dis


def _tc_last(hs2_ref, p_ref, degp_ref, b2_ref, batch_ref, wfc_ref, bfc_ref,
             out_ref):
    dis = _dis_from_degp(degp_ref)
    ssum = (hs2_ref[...] + p_ref[0, 0:N, :] + p_ref[1, 0:N, :])
    r2 = jnp.maximum(dis * ssum + b2_ref[...], 0.0)
    gid = lax.broadcasted_iota(jnp.int32, (N, G), 1)
    oneh = (batch_ref[...] == gid).astype(jnp.float32)
    sums = lax.dot_general(oneh, r2, (((0,), (0,)), ((), ())),
                           preferred_element_type=jnp.float32,
                           precision=lax.Precision.HIGHEST)
    ones_col = jnp.ones((N, 1), jnp.float32)
    cnts = lax.dot_general(oneh, ones_col, (((0,), (0,)), ((), ())),
                           preferred_element_type=jnp.float32,
                           precision=lax.Precision.HIGHEST)
    g = sums / jnp.maximum(cnts, 1.0)
    out_ref[...] = lax.dot_general(g, wfc_ref[...], (((1,), (0,)), ((), ())),
                                   preferred_element_type=jnp.float32,
                                   precision=lax.Precision.HIGHEST) + bfc_ref[...]


_matmul1 = pl.pallas_call(
    _tc_matmul1, out_shape=jax.ShapeDtypeStruct((N, D_H), jnp.float32))
_scale1 = pl.pallas_call(
    _tc_scale1, out_shape=jax.ShapeDtypeStruct((N, D_H), jnp.float32))
_mid = pl.pallas_call(
    _tc_mid, out_shape=jax.ShapeDtypeStruct((N, D_H), jnp.float32))
_last = pl.pallas_call(
    _tc_last, out_shape=jax.ShapeDtypeStruct((G, 1), jnp.float32))


def kernel(x, edge_index, batch, W1, b1, W2, b2, Wfc, bfc):
    ei = edge_index.astype(jnp.int32)
    degp = _sc_deg(ei)
    h1 = _matmul1(x, W1)
    hs1 = _scale1(h1, degp)
    p1 = _sc_agg(hs1, ei)
    hs2 = _mid(hs1, p1, degp, b1.reshape(1, D_H), W2)
    p2 = _sc_agg(hs2, ei)
    out = _last(hs2, p2, degp, b2.reshape(1, D_H),
                batch.astype(jnp.int32).reshape(N, 1), Wfc,
                bfc.reshape(1, 1))
    return out.reshape(G)
